# baseline (speedup 1.0000x reference)
import functools

import jax
import jax.numpy as jnp
import numpy as np
from jax import lax
from jax.experimental import pallas as pl
from jax.experimental.pallas import tpu as pltpu
from jax.experimental.pallas import tpu_sc as plsc

N = 10000
E = 320000
D = 128
DW = D // 2

NC = 2
NS = 16
LANES = 16

NW = NC * NS
EDGES_PER_W = E // NW
CHUNK = 80
NCHUNKS = 128
NCHUNKS_PAD = 136
NROW_CHUNKS = N // CHUNK

_PERM = np.concatenate(
    [np.concatenate([32 * q + np.arange(0, 32, 2),
                     32 * q + np.arange(1, 32, 2)]) for q in range(4)])

_DNUMS = lax.GatherDimensionNumbers(
    offset_dims=(), collapsed_slice_dims=(0,), start_index_map=(0,))


def _sc_body(x_hbm, rows_hbm, cols_hbm, vals_hbm, out_hbm,
             colsA, rowsA, valsA, colsB, rowsB, valsB,
             colsC, rowsC, valsC, colsD, rowsD, valsD,
             gbuf0, gbuf1, sbuf, agg,
             gsem0, gsem1, isemA, isemB, isemC, isemD):
    c = lax.axis_index("c")
    s = lax.axis_index("s")
    w = c * NS + s

    sets = [(colsA, rowsA, valsA, isemA), (colsB, rowsB, valsB, isemB),
            (colsC, rowsC, valsC, isemC), (colsD, rowsD, valsD, isemD)]
    gbs = [(gbuf0, gsem0), (gbuf1, gsem1)]

    zero16 = jnp.zeros((LANES,), jnp.float32)

    def _zrow(r, carry):
        for k in range(D // LANES):
            sbuf[r, pl.ds(k * LANES, LANES)] = zero16
        return carry

    lax.fori_loop(0, CHUNK, _zrow, 0)

    for i in range((NROW_CHUNKS + NS - 1) // NS):
        cid = s + i * NS

        @pl.when(cid < NROW_CHUNKS)
        def _zero_chunk():
            r0 = pl.multiple_of(cid * CHUNK, CHUNK)
            pltpu.sync_copy(sbuf, agg.at[pl.ds(r0, CHUNK)])

    plsc.subcore_barrier()

    def i_start(j, st):
        cb, rb, vb, sem = st
        pltpu.async_copy(cols_hbm.at[w, pl.ds(j, 1)], cb, sem)
        pltpu.async_copy(rows_hbm.at[w, pl.ds(j, 1)], rb, sem)
        pltpu.async_copy(vals_hbm.at[w, pl.ds(j, 1)], vb, sem)

    def i_wait(j, st):
        cb, rb, vb, sem = st
        pltpu.make_async_copy(cols_hbm.at[w, pl.ds(j, 1)], cb, sem).wait()
        pltpu.make_async_copy(rows_hbm.at[w, pl.ds(j, 1)], rb, sem).wait()
        pltpu.make_async_copy(vals_hbm.at[w, pl.ds(j, 1)], vb, sem).wait()

    def g_start(st, buf, sem):
        pltpu.async_copy(x_hbm.at[st[0].at[0]], buf, sem)

    def g_wait(st, buf, sem):
        pltpu.make_async_copy(x_hbm.at[st[0].at[0]], buf, sem).wait()

    mask_hi = jnp.full((LANES,), -65536, jnp.int32)

    def scale_convert(gb, vb):
        def grp(g, carry):
            vv = vb[0, pl.ds(g * LANES, LANES)]
            for jj in range(LANES):
                bc = lax.gather(
                    vv, jnp.full((LANES, 1), jj, jnp.int32), _DNUMS, (1,),
                    mode=lax.GatherScatterMode.PROMISE_IN_BOUNDS)
                r = g * LANES + jj
                for q in range(DW // LANES):
                    wi = gb[r, pl.ds(q * LANES, LANES)]
                    lo = plsc.bitcast(wi << 16, jnp.float32)
                    hi = plsc.bitcast(wi & mask_hi, jnp.float32)
                    sbuf[r, pl.ds(32 * q, LANES)] = lo * bc
                    sbuf[r, pl.ds(32 * q + LANES, LANES)] = hi * bc
            return carry

        lax.fori_loop(0, CHUNK // LANES, grp, 0)

    i_start(0, sets[0])
    i_start(1, sets[1])
    i_wait(0, sets[0])
    i_wait(1, sets[1])
    g_start(sets[0], gbuf0, gsem0)
    g_start(sets[1], gbuf1, gsem1)
    i_start(2, sets[2])
    i_start(3, sets[3])

    def _quad(t, carry):
        j = 4 * t
        for k in range(4):
            cur = sets[k]
            nxt = sets[(k + 2) % 4]
            gb, gs = gbs[k % 2]
            g_wait(cur, gb, gs)
            scale_convert(gb, cur[2])
            pltpu.sync_copy(sbuf, agg.at[cur[1].at[0]], add=True)
            i_wait(j + k + 2, nxt)
            g_start(nxt, gb, gs)
            i_start(j + k + 4, cur)
        return carry

    lax.fori_loop(0, NCHUNKS // 4, _quad, 0)

    g_wait(sets[0], gbuf0, gsem0)
    g_wait(sets[1], gbuf1, gsem1)
    i_wait(NCHUNKS + 2, sets[2])
    i_wait(NCHUNKS + 3, sets[3])

    plsc.subcore_barrier()

    for i in range((NROW_CHUNKS + NS - 1) // NS):
        cid = s + i * NS

        @pl.when(cid < NROW_CHUNKS)
        def _copy_chunk():
            r0 = pl.multiple_of(cid * CHUNK, CHUNK)
            pltpu.sync_copy(agg.at[pl.ds(r0, CHUNK)],
                            out_hbm.at[c, pl.ds(r0, CHUNK)])


_idx_vmem = lambda: pltpu.VMEM((1, CHUNK), jnp.int32)
_val_vmem = lambda: pltpu.VMEM((1, CHUNK), jnp.float32)

_sc_segment_sum = functools.partial(
    pl.kernel,
    out_type=jax.ShapeDtypeStruct((NC, N, D), jnp.float32),
    mesh=plsc.VectorSubcoreMesh(core_axis_name="c", subcore_axis_name="s"),
    compiler_params=pltpu.CompilerParams(needs_layout_passes=False,
                                         use_tc_tiling_on_sc=False),
    scratch_types=[
        _idx_vmem(), _idx_vmem(), _val_vmem(),
        _idx_vmem(), _idx_vmem(), _val_vmem(),
        _idx_vmem(), _idx_vmem(), _val_vmem(),
        _idx_vmem(), _idx_vmem(), _val_vmem(),
        pltpu.VMEM((CHUNK, DW), jnp.int32),
        pltpu.VMEM((CHUNK, DW), jnp.int32),
        pltpu.VMEM((CHUNK, D), jnp.float32),
        pltpu.VMEM_SHARED((N, D), jnp.float32),
        pltpu.SemaphoreType.DMA,
        pltpu.SemaphoreType.DMA,
        pltpu.SemaphoreType.DMA,
        pltpu.SemaphoreType.DMA,
        pltpu.SemaphoreType.DMA,
        pltpu.SemaphoreType.DMA,
    ],
)(_sc_body)


BLK = 1000


def _tc_linear_body(p0_ref, p1_ref, wt_ref, b_ref, o_ref):
    acc = p0_ref[...] + p1_ref[...]
    o_ref[...] = (
        jnp.dot(acc, wt_ref[...], preferred_element_type=jnp.float32)
        + b_ref[...]
    )


def _tc_linear(p0, p1, wt, b2):
    return pl.pallas_call(
        _tc_linear_body,
        grid=(N // BLK,),
        in_specs=[
            pl.BlockSpec((BLK, D), lambda i: (i, 0)),
            pl.BlockSpec((BLK, D), lambda i: (i, 0)),
            pl.BlockSpec((D, D), lambda i: (0, 0)),
            pl.BlockSpec((1, D), lambda i: (0, 0)),
        ],
        out_specs=pl.BlockSpec((BLK, D), lambda i: (i, 0)),
        out_shape=jax.ShapeDtypeStruct((N, D), jnp.float32),
    )(p0, p1, wt, b2)


def kernel(X, L_rows, L_cols, L_vals, W, b):
    pad = NCHUNKS_PAD * CHUNK - EDGES_PER_W
    rows3 = jnp.pad(L_rows.reshape(NW, EDGES_PER_W),
                    ((0, 0), (0, pad))).reshape(NW, NCHUNKS_PAD, CHUNK)
    cols3 = jnp.pad(L_cols.reshape(NW, EDGES_PER_W),
                    ((0, 0), (0, pad))).reshape(NW, NCHUNKS_PAD, CHUNK)
    vals3 = jnp.pad(L_vals.reshape(NW, EDGES_PER_W),
                    ((0, 0), (0, pad))).reshape(NW, NCHUNKS_PAD, CHUNK)
    xp = jax.lax.bitcast_convert_type(
        X.astype(jnp.bfloat16).reshape(N, DW, 2), jnp.int32)
    partials = _sc_segment_sum(xp, rows3, cols3, vals3)
    wtp = W.T[_PERM]
    return _tc_linear(partials[0], partials[1], wtp, b.reshape(1, D))


# speedup vs baseline: 1.2002x; 1.2002x over previous
import functools

import jax
import jax.numpy as jnp
from jax import lax
from jax.experimental import pallas as pl
from jax.experimental.pallas import tpu as pltpu
from jax.experimental.pallas import tpu_sc as plsc

N = 10000
E = 320000
D = 128
DH = D // 2

NC = 2
NS = 16
LANES = 16

NW = NC * NS
EDGES_PER_W = E // NW
CHUNK = 80
NCHUNKS = 128
NCHUNKS_PAD = 136
NROW_CHUNKS = N // CHUNK

_DNUMS = lax.GatherDimensionNumbers(
    offset_dims=(), collapsed_slice_dims=(0,), start_index_map=(0,))


def _sc_body(x0_hbm, x1_hbm, rows_hbm, cols_hbm, vals_hbm, out_hbm,
             colsA, rowsA, valsA, colsB, rowsB, valsB,
             colsC, rowsC, valsC, colsD, rowsD, valsD,
             gbuf0, gbuf1, x_stage, agg,
             gsem0, gsem1, isemA, isemB, isemC, isemD):
    c = lax.axis_index("c")
    s = lax.axis_index("s")
    w = c * NS + s

    sets = [(colsA, rowsA, valsA, isemA), (colsB, rowsB, valsB, isemB),
            (colsC, rowsC, valsC, isemC), (colsD, rowsD, valsD, isemD)]
    gbs = [(gbuf0, gsem0), (gbuf1, gsem1)]

    def i_start(j, st):
        cb, rb, vb, sem = st
        pltpu.async_copy(cols_hbm.at[w, j], cb, sem)
        pltpu.async_copy(rows_hbm.at[w, j], rb, sem)
        pltpu.async_copy(vals_hbm.at[w, j], vb, sem)

    def i_wait(j, st):
        cb, rb, vb, sem = st
        pltpu.make_async_copy(cols_hbm.at[w, j], cb, sem).wait()
        pltpu.make_async_copy(rows_hbm.at[w, j], rb, sem).wait()
        pltpu.make_async_copy(vals_hbm.at[w, j], vb, sem).wait()

    def g_start(st, buf, sem):
        pltpu.async_copy(x_stage.at[st[0]], buf, sem)

    def g_wait(st, buf, sem):
        pltpu.make_async_copy(x_stage.at[st[0]], buf, sem).wait()

    zero16 = jnp.zeros((LANES,), jnp.float32)

    def scale(buf, vb):
        def grp(g, carry):
            vv = vb[pl.ds(g * LANES, LANES)]
            for jj in range(LANES):
                bc = lax.gather(
                    vv, jnp.full((LANES, 1), jj, jnp.int32), _DNUMS, (1,),
                    mode=lax.GatherScatterMode.PROMISE_IN_BOUNDS)
                r = g * LANES + jj
                for k in range(DH // LANES):
                    sl = pl.ds(k * LANES, LANES)
                    buf[r, sl] = buf[r, sl] * bc
            return carry

        lax.fori_loop(0, CHUNK // LANES, grp, 0)

    for h, xh_hbm in enumerate((x0_hbm, x1_hbm)):
        def _zrow(r, carry):
            for k in range(DH // LANES):
                gbuf0[r, pl.ds(k * LANES, LANES)] = zero16
            return carry

        lax.fori_loop(0, CHUNK, _zrow, 0)

        for i in range((NROW_CHUNKS + NS - 1) // NS):
            cid = s + i * NS

            @pl.when(cid < NROW_CHUNKS)
            def _prep_chunk():
                r0 = pl.multiple_of(cid * CHUNK, CHUNK)
                pltpu.sync_copy(xh_hbm.at[pl.ds(r0, CHUNK)],
                                x_stage.at[pl.ds(r0, CHUNK)])
                pltpu.sync_copy(gbuf0, agg.at[pl.ds(r0, CHUNK)])

        plsc.subcore_barrier()

        i_start(0, sets[0])
        i_start(1, sets[1])
        i_wait(0, sets[0])
        i_wait(1, sets[1])
        g_start(sets[0], gbuf0, gsem0)
        g_start(sets[1], gbuf1, gsem1)
        i_start(2, sets[2])
        i_start(3, sets[3])

        def _quad(t, carry):
            j = 4 * t
            for k in range(4):
                cur = sets[k]
                nxt = sets[(k + 2) % 4]
                gb, gs = gbs[k % 2]
                g_wait(cur, gb, gs)
                scale(gb, cur[2])
                pltpu.sync_copy(gb, agg.at[cur[1]], add=True)
                i_wait(j + k + 2, nxt)
                g_start(nxt, gb, gs)
                i_start(j + k + 4, cur)
            return carry

        lax.fori_loop(0, NCHUNKS // 4, _quad, 0)

        g_wait(sets[0], gbuf0, gsem0)
        g_wait(sets[1], gbuf1, gsem1)
        i_wait(NCHUNKS + 2, sets[2])
        i_wait(NCHUNKS + 3, sets[3])

        plsc.subcore_barrier()

        for i in range((NROW_CHUNKS + NS - 1) // NS):
            cid = s + i * NS

            @pl.when(cid < NROW_CHUNKS)
            def _copy_chunk():
                r0 = pl.multiple_of(cid * CHUNK, CHUNK)
                pltpu.sync_copy(agg.at[pl.ds(r0, CHUNK)],
                                out_hbm.at[c, h, pl.ds(r0, CHUNK)])


_idx_vmem = lambda: pltpu.VMEM((CHUNK,), jnp.int32)
_val_vmem = lambda: pltpu.VMEM((CHUNK,), jnp.float32)

_sc_segment_sum = functools.partial(
    pl.kernel,
    out_type=jax.ShapeDtypeStruct((NC, 2, N, DH), jnp.float32),
    mesh=plsc.VectorSubcoreMesh(core_axis_name="c", subcore_axis_name="s"),
    compiler_params=pltpu.CompilerParams(use_tc_tiling_on_sc=False),
    scratch_types=[
        _idx_vmem(), _idx_vmem(), _val_vmem(),
        _idx_vmem(), _idx_vmem(), _val_vmem(),
        _idx_vmem(), _idx_vmem(), _val_vmem(),
        _idx_vmem(), _idx_vmem(), _val_vmem(),
        pltpu.VMEM((CHUNK, DH), jnp.float32),
        pltpu.VMEM((CHUNK, DH), jnp.float32),
        pltpu.VMEM_SHARED((N, DH), jnp.float32),
        pltpu.VMEM_SHARED((N, DH), jnp.float32),
        pltpu.SemaphoreType.DMA,
        pltpu.SemaphoreType.DMA,
        pltpu.SemaphoreType.DMA,
        pltpu.SemaphoreType.DMA,
        pltpu.SemaphoreType.DMA,
        pltpu.SemaphoreType.DMA,
    ],
)(_sc_body)


BLK = 1000


def _tc_linear_body(p00_ref, p01_ref, p10_ref, p11_ref,
                    wt0_ref, wt1_ref, b_ref, o_ref):
    acc0 = p00_ref[...] + p10_ref[...]
    acc1 = p01_ref[...] + p11_ref[...]
    o_ref[...] = (
        jnp.dot(acc0, wt0_ref[...], preferred_element_type=jnp.float32)
        + jnp.dot(acc1, wt1_ref[...], preferred_element_type=jnp.float32)
        + b_ref[...]
    )


def _tc_linear(p00, p01, p10, p11, wt0, wt1, b2):
    return pl.pallas_call(
        _tc_linear_body,
        grid=(N // BLK,),
        in_specs=[
            pl.BlockSpec((BLK, DH), lambda i: (i, 0)),
            pl.BlockSpec((BLK, DH), lambda i: (i, 0)),
            pl.BlockSpec((BLK, DH), lambda i: (i, 0)),
            pl.BlockSpec((BLK, DH), lambda i: (i, 0)),
            pl.BlockSpec((DH, D), lambda i: (0, 0)),
            pl.BlockSpec((DH, D), lambda i: (0, 0)),
            pl.BlockSpec((1, D), lambda i: (0, 0)),
        ],
        out_specs=pl.BlockSpec((BLK, D), lambda i: (i, 0)),
        out_shape=jax.ShapeDtypeStruct((N, D), jnp.float32),
    )(p00, p01, p10, p11, wt0, wt1, b2)


def kernel(X, L_rows, L_cols, L_vals, W, b):
    pad = NCHUNKS_PAD * CHUNK - EDGES_PER_W
    rows3 = jnp.pad(L_rows.reshape(NW, EDGES_PER_W),
                    ((0, 0), (0, pad))).reshape(NW, NCHUNKS_PAD, CHUNK)
    cols3 = jnp.pad(L_cols.reshape(NW, EDGES_PER_W),
                    ((0, 0), (0, pad))).reshape(NW, NCHUNKS_PAD, CHUNK)
    vals3 = jnp.pad(L_vals.reshape(NW, EDGES_PER_W),
                    ((0, 0), (0, pad))).reshape(NW, NCHUNKS_PAD, CHUNK)
    x0 = X[:, :DH]
    x1 = X[:, DH:]
    partials = _sc_segment_sum(x0, x1, rows3, cols3, vals3)
    wt = W.T
    return _tc_linear(partials[0, 0], partials[0, 1],
                      partials[1, 0], partials[1, 1],
                      wt[:DH], wt[DH:], b.reshape(1, D))


# speedup vs baseline: 1.6568x; 1.3804x over previous
import functools

import jax
import jax.numpy as jnp
from jax import lax
from jax.experimental import pallas as pl
from jax.experimental.pallas import tpu as pltpu
from jax.experimental.pallas import tpu_sc as plsc

N = 10000
E = 320000
D = 128

NC = 2
NS = 16
LANES = 16

NW = NC * NS
EDGES_PER_W = E // NW
CHUNK = 80
NCHUNKS = EDGES_PER_W // CHUNK
NROW_CHUNKS = N // CHUNK

_DNUMS = lax.GatherDimensionNumbers(
    offset_dims=(), collapsed_slice_dims=(0,), start_index_map=(0,))


def _sc_body(x_hbm, idx_hbm, out_hbm, ibuf, gbuf, agg):
    c = lax.axis_index("c")
    s = lax.axis_index("s")
    w = c * NS + s

    zero16 = jnp.zeros((LANES,), jnp.float32)

    def _zrow(r, carry):
        for k in range(D // LANES):
            gbuf[r, pl.ds(k * LANES, LANES)] = zero16
        return carry

    lax.fori_loop(0, CHUNK, _zrow, 0)

    for i in range((NROW_CHUNKS + NS - 1) // NS):
        cid = s + i * NS

        @pl.when(cid < NROW_CHUNKS)
        def _zero_chunk():
            r0 = pl.multiple_of(cid * CHUNK, CHUNK)
            pltpu.sync_copy(gbuf, agg.at[pl.ds(r0, CHUNK)])

    plsc.subcore_barrier()

    def _chunk(j, carry):
        pltpu.sync_copy(idx_hbm.at[w, j], ibuf)
        pltpu.sync_copy(x_hbm.at[ibuf.at[0]], gbuf)
        for g in range(CHUNK // LANES):
            vv = plsc.bitcast(ibuf[2, pl.ds(g * LANES, LANES)], jnp.float32)
            for jj in range(LANES):
                bc = lax.gather(
                    vv, jnp.full((LANES, 1), jj, jnp.int32), _DNUMS, (1,),
                    mode=lax.GatherScatterMode.PROMISE_IN_BOUNDS)
                r = g * LANES + jj
                for k in range(D // LANES):
                    sl = pl.ds(k * LANES, LANES)
                    gbuf[r, sl] = gbuf[r, sl] * bc
        pltpu.sync_copy(gbuf, agg.at[ibuf.at[1]], add=True)
        return carry

    lax.fori_loop(0, NCHUNKS, _chunk, 0)
    plsc.subcore_barrier()

    for i in range((NROW_CHUNKS + NS - 1) // NS):
        cid = s + i * NS

        @pl.when(cid < NROW_CHUNKS)
        def _copy_chunk():
            r0 = pl.multiple_of(cid * CHUNK, CHUNK)
            pltpu.sync_copy(agg.at[pl.ds(r0, CHUNK)],
                            out_hbm.at[c, pl.ds(r0, CHUNK)])


_sc_segment_sum = functools.partial(
    pl.kernel,
    out_type=jax.ShapeDtypeStruct((NC, N, D), jnp.float32),
    mesh=plsc.VectorSubcoreMesh(core_axis_name="c", subcore_axis_name="s"),
    compiler_params=pltpu.CompilerParams(needs_layout_passes=False),
    scratch_types=[
        pltpu.VMEM((3, CHUNK), jnp.int32),
        pltpu.VMEM((CHUNK, D), jnp.float32),
        pltpu.VMEM_SHARED((N, D), jnp.float32),
    ],
)(_sc_body)


BLK = 1000


def _tc_linear_body(p0_ref, p1_ref, wt_ref, b_ref, o_ref):
    acc = p0_ref[...] + p1_ref[...]
    o_ref[...] = (
        jnp.dot(acc, wt_ref[...], preferred_element_type=jnp.float32)
        + b_ref[...]
    )


def _tc_linear(p0, p1, wt, b2):
    return pl.pallas_call(
        _tc_linear_body,
        grid=(N // BLK,),
        in_specs=[
            pl.BlockSpec((BLK, D), lambda i: (i, 0)),
            pl.BlockSpec((BLK, D), lambda i: (i, 0)),
            pl.BlockSpec((D, D), lambda i: (0, 0)),
            pl.BlockSpec((1, D), lambda i: (0, 0)),
        ],
        out_specs=pl.BlockSpec((BLK, D), lambda i: (i, 0)),
        out_shape=jax.ShapeDtypeStruct((N, D), jnp.float32),
    )(p0, p1, wt, b2)


def kernel(X, L_rows, L_cols, L_vals, W, b):
    cols3 = L_cols.reshape(NW, NCHUNKS, 1, CHUNK)
    rows3 = L_rows.reshape(NW, NCHUNKS, 1, CHUNK)
    vals3 = jax.lax.bitcast_convert_type(
        L_vals.reshape(NW, NCHUNKS, 1, CHUNK), jnp.int32)
    idx = jnp.concatenate([cols3, rows3, vals3], axis=2)
    partials = _sc_segment_sum(X, idx)
    return _tc_linear(partials[0], partials[1], W.T, b.reshape(1, D))


# speedup vs baseline: 1.9269x; 1.1630x over previous
import functools

import jax
import jax.numpy as jnp
from jax import lax
from jax.experimental import pallas as pl
from jax.experimental.pallas import tpu as pltpu
from jax.experimental.pallas import tpu_sc as plsc

N = 10000
E = 320000
D = 128

NC = 2
NS = 16
LANES = 16

NW = NC * NS
EDGES_PER_W = E // NW
CHUNK = 80
NCHUNKS = EDGES_PER_W // CHUNK
FUSE = 5
NROW_CHUNKS = N // CHUNK

_DNUMS = lax.GatherDimensionNumbers(
    offset_dims=(), collapsed_slice_dims=(0,), start_index_map=(0,))


def _sc_body(x_hbm, idx_hbm, out_hbm, ibuf, gbuf, agg):
    c = lax.axis_index("c")
    s = lax.axis_index("s")
    w = c * NS + s

    zero16 = jnp.zeros((LANES,), jnp.float32)

    def _zrow(r, carry):
        for k in range(D // LANES):
            gbuf[r, pl.ds(k * LANES, LANES)] = zero16
        return carry

    lax.fori_loop(0, CHUNK, _zrow, 0)

    for i in range((NROW_CHUNKS + NS - 1) // NS):
        cid = s + i * NS

        @pl.when(cid < NROW_CHUNKS)
        def _zero_chunk():
            r0 = pl.multiple_of(cid * CHUNK, CHUNK)
            pltpu.sync_copy(gbuf, agg.at[pl.ds(r0, CHUNK)])

    plsc.subcore_barrier()

    def _chunk(j, carry):
        u = j // FUSE
        q = j - u * FUSE

        @pl.when(q == 0)
        def _load_idx():
            pltpu.sync_copy(idx_hbm.at[w, u], ibuf)

        pltpu.sync_copy(x_hbm.at[ibuf.at[3 * q]], gbuf)
        for g in range(CHUNK // LANES):
            vv = plsc.bitcast(
                ibuf[3 * q + 2, pl.ds(g * LANES, LANES)], jnp.float32)
            for jj in range(LANES):
                bc = lax.gather(
                    vv, jnp.full((LANES, 1), jj, jnp.int32), _DNUMS, (1,),
                    mode=lax.GatherScatterMode.PROMISE_IN_BOUNDS)
                r = g * LANES + jj
                for k in range(D // LANES):
                    sl = pl.ds(k * LANES, LANES)
                    gbuf[r, sl] = gbuf[r, sl] * bc
        pltpu.sync_copy(gbuf, agg.at[ibuf.at[3 * q + 1]], add=True)
        return carry

    lax.fori_loop(0, NCHUNKS, _chunk, 0)
    plsc.subcore_barrier()

    for i in range((NROW_CHUNKS + NS - 1) // NS):
        cid = s + i * NS

        @pl.when(cid < NROW_CHUNKS)
        def _copy_chunk():
            r0 = pl.multiple_of(cid * CHUNK, CHUNK)
            pltpu.sync_copy(agg.at[pl.ds(r0, CHUNK)],
                            out_hbm.at[c, pl.ds(r0, CHUNK)])


_sc_segment_sum = functools.partial(
    pl.kernel,
    out_type=jax.ShapeDtypeStruct((NC, N, D), jnp.float32),
    mesh=plsc.VectorSubcoreMesh(core_axis_name="c", subcore_axis_name="s"),
    compiler_params=pltpu.CompilerParams(needs_layout_passes=False),
    scratch_types=[
        pltpu.VMEM((3 * FUSE, CHUNK), jnp.int32),
        pltpu.VMEM((CHUNK, D), jnp.float32),
        pltpu.VMEM_SHARED((N, D), jnp.float32),
    ],
)(_sc_body)


BLK = 1000


def _tc_linear_body(p0_ref, p1_ref, wt_ref, b_ref, o_ref):
    acc = p0_ref[...] + p1_ref[...]
    o_ref[...] = (
        jnp.dot(acc, wt_ref[...], preferred_element_type=jnp.float32)
        + b_ref[...]
    )


def _tc_linear(p0, p1, wt, b2):
    return pl.pallas_call(
        _tc_linear_body,
        grid=(N // BLK,),
        in_specs=[
            pl.BlockSpec((BLK, D), lambda i: (i, 0)),
            pl.BlockSpec((BLK, D), lambda i: (i, 0)),
            pl.BlockSpec((D, D), lambda i: (0, 0)),
            pl.BlockSpec((1, D), lambda i: (0, 0)),
        ],
        out_specs=pl.BlockSpec((BLK, D), lambda i: (i, 0)),
        out_shape=jax.ShapeDtypeStruct((N, D), jnp.float32),
    )(p0, p1, wt, b2)


def kernel(X, L_rows, L_cols, L_vals, W, b):
    nsup = NCHUNKS // FUSE
    cols4 = L_cols.reshape(NW, nsup, FUSE, 1, CHUNK)
    rows4 = L_rows.reshape(NW, nsup, FUSE, 1, CHUNK)
    vals4 = jax.lax.bitcast_convert_type(
        L_vals.reshape(NW, nsup, FUSE, 1, CHUNK), jnp.int32)
    idx = jnp.concatenate([cols4, rows4, vals4], axis=3).reshape(
        NW, nsup, 3 * FUSE, CHUNK)
    partials = _sc_segment_sum(X, idx)
    return _tc_linear(partials[0], partials[1], W.T, b.reshape(1, D))


# speedup vs baseline: 2.6286x; 1.3642x over previous
import functools

import jax
import jax.numpy as jnp
from jax import lax
from jax.experimental import pallas as pl
from jax.experimental.pallas import tpu as pltpu
from jax.experimental.pallas import tpu_sc as plsc

N = 10000
E = 320000
D = 128

NC = 2
NS = 16
LANES = 16

NW = NC * NS
EDGES_PER_W = E // NW
CHUNK = 80
NCHUNKS = EDGES_PER_W // CHUNK
FUSE = 5
NROW_CHUNKS = N // CHUNK

_DNUMS = lax.GatherDimensionNumbers(
    offset_dims=(), collapsed_slice_dims=(0,), start_index_map=(0,))


def _sc_body(x_hbm, idx_hbm, out_hbm, ibuf, gbuf, gbuf1, agg, gsem0, gsem1):
    c = lax.axis_index("c")
    s = lax.axis_index("s")
    w = c * NS + s

    zero16 = jnp.zeros((LANES,), jnp.float32)

    def _zrow(r, carry):
        for k in range(D // LANES):
            gbuf[r, pl.ds(k * LANES, LANES)] = zero16
        return carry

    lax.fori_loop(0, CHUNK, _zrow, 0)

    for i in range((NROW_CHUNKS + NS - 1) // NS):
        cid = s + i * NS

        @pl.when(cid < NROW_CHUNKS)
        def _zero_chunk():
            r0 = pl.multiple_of(cid * CHUNK, CHUNK)
            pltpu.sync_copy(gbuf, agg.at[pl.ds(r0, CHUNK)])

    plsc.subcore_barrier()

    def scale(gb, vrow):
        def grp(g, carry):
            vv = plsc.bitcast(
                ibuf[vrow, pl.ds(g * LANES, LANES)], jnp.float32)
            for jj in range(LANES):
                bc = lax.gather(
                    vv, jnp.full((LANES, 1), jj, jnp.int32), _DNUMS, (1,),
                    mode=lax.GatherScatterMode.PROMISE_IN_BOUNDS)
                r = g * LANES + jj
                for k in range(D // LANES):
                    sl = pl.ds(k * LANES, LANES)
                    gb[r, sl] = gb[r, sl] * bc
            return carry

        lax.fori_loop(0, CHUNK // LANES, grp, 0)

    def _super(u, carry):
        pltpu.sync_copy(idx_hbm.at[w, u], ibuf)
        pltpu.async_copy(x_hbm.at[ibuf.at[0]], gbuf, gsem0)
        for k in range(FUSE):
            gb, gs = (gbuf, gsem0) if k % 2 == 0 else (gbuf1, gsem1)
            if k + 1 < FUSE:
                nb, ns = (gbuf, gsem0) if k % 2 == 1 else (gbuf1, gsem1)
                pltpu.async_copy(x_hbm.at[ibuf.at[3 * (k + 1)]], nb, ns)
            pltpu.make_async_copy(x_hbm.at[ibuf.at[3 * k]], gb, gs).wait()
            scale(gb, 3 * k + 2)
            pltpu.sync_copy(gb, agg.at[ibuf.at[3 * k + 1]], add=True)
        return carry

    lax.fori_loop(0, NCHUNKS // FUSE, _super, 0)
    plsc.subcore_barrier()

    for i in range((NROW_CHUNKS + NS - 1) // NS):
        cid = s + i * NS

        @pl.when(cid < NROW_CHUNKS)
        def _copy_chunk():
            r0 = pl.multiple_of(cid * CHUNK, CHUNK)
            pltpu.sync_copy(agg.at[pl.ds(r0, CHUNK)],
                            out_hbm.at[c, pl.ds(r0, CHUNK)])


_sc_segment_sum = functools.partial(
    pl.kernel,
    out_type=jax.ShapeDtypeStruct((NC, N, D), jnp.float32),
    mesh=plsc.VectorSubcoreMesh(core_axis_name="c", subcore_axis_name="s"),
    compiler_params=pltpu.CompilerParams(needs_layout_passes=False),
    scratch_types=[
        pltpu.VMEM((3 * FUSE, CHUNK), jnp.int32),
        pltpu.VMEM((CHUNK, D), jnp.float32),
        pltpu.VMEM((CHUNK, D), jnp.float32),
        pltpu.VMEM_SHARED((N, D), jnp.float32),
        pltpu.SemaphoreType.DMA,
        pltpu.SemaphoreType.DMA,
    ],
)(_sc_body)


BLK = 1000


def _tc_linear_body(p0_ref, p1_ref, wt_ref, b_ref, o_ref):
    acc = p0_ref[...] + p1_ref[...]
    o_ref[...] = (
        jnp.dot(acc, wt_ref[...], preferred_element_type=jnp.float32)
        + b_ref[...]
    )


def _tc_linear(p0, p1, wt, b2):
    return pl.pallas_call(
        _tc_linear_body,
        grid=(N // BLK,),
        in_specs=[
            pl.BlockSpec((BLK, D), lambda i: (i, 0)),
            pl.BlockSpec((BLK, D), lambda i: (i, 0)),
            pl.BlockSpec((D, D), lambda i: (0, 0)),
            pl.BlockSpec((1, D), lambda i: (0, 0)),
        ],
        out_specs=pl.BlockSpec((BLK, D), lambda i: (i, 0)),
        out_shape=jax.ShapeDtypeStruct((N, D), jnp.float32),
    )(p0, p1, wt, b2)


def kernel(X, L_rows, L_cols, L_vals, W, b):
    nsup = NCHUNKS // FUSE
    cols4 = L_cols.reshape(NW, nsup, FUSE, 1, CHUNK)
    rows4 = L_rows.reshape(NW, nsup, FUSE, 1, CHUNK)
    vals4 = jax.lax.bitcast_convert_type(
        L_vals.reshape(NW, nsup, FUSE, 1, CHUNK), jnp.int32)
    idx = jnp.concatenate([cols4, rows4, vals4], axis=3).reshape(
        NW, nsup, 3 * FUSE, CHUNK)
    partials = _sc_segment_sum(X, idx)
    return _tc_linear(partials[0], partials[1], W.T, b.reshape(1, D))


# speedup vs baseline: 2.8376x; 1.0795x over previous
import functools

import jax
import jax.numpy as jnp
from jax import lax
from jax.experimental import pallas as pl
from jax.experimental.pallas import tpu as pltpu
from jax.experimental.pallas import tpu_sc as plsc

N = 10000
E = 320000
D = 128

NC = 2
NS = 16
LANES = 16

NW = NC * NS
EDGES_PER_W = E // NW
CHUNK = 80
NCHUNKS = EDGES_PER_W // CHUNK
FUSE = 5
NROW_CHUNKS = N // CHUNK

_DNUMS = lax.GatherDimensionNumbers(
    offset_dims=(), collapsed_slice_dims=(0,), start_index_map=(0,))


def _sc_body(x_hbm, idx_hbm, out_hbm, ibuf, gbuf, gbuf1, gbuf2, agg,
             gsem0, gsem1, gsem2):
    c = lax.axis_index("c")
    s = lax.axis_index("s")
    w = c * NS + s

    zero16 = jnp.zeros((LANES,), jnp.float32)

    def _zrow(r, carry):
        for k in range(D // LANES):
            gbuf[r, pl.ds(k * LANES, LANES)] = zero16
        return carry

    lax.fori_loop(0, CHUNK, _zrow, 0)

    for i in range((NROW_CHUNKS + NS - 1) // NS):
        cid = s + i * NS

        @pl.when(cid < NROW_CHUNKS)
        def _zero_chunk():
            r0 = pl.multiple_of(cid * CHUNK, CHUNK)
            pltpu.sync_copy(gbuf, agg.at[pl.ds(r0, CHUNK)])

    plsc.subcore_barrier()

    def scale(gb, vrow):
        def grp(g, carry):
            vv = plsc.bitcast(
                ibuf[vrow, pl.ds(g * LANES, LANES)], jnp.float32)
            for jj in range(LANES):
                bc = lax.gather(
                    vv, jnp.full((LANES, 1), jj, jnp.int32), _DNUMS, (1,),
                    mode=lax.GatherScatterMode.PROMISE_IN_BOUNDS)
                r = g * LANES + jj
                for k in range(D // LANES):
                    sl = pl.ds(k * LANES, LANES)
                    gb[r, sl] = gb[r, sl] * bc
            return carry

        lax.fori_loop(0, CHUNK // LANES, grp, 0)

    bufs = [(gbuf, gsem0), (gbuf1, gsem1), (gbuf2, gsem2)]

    def _super(u, carry):
        pltpu.sync_copy(idx_hbm.at[w, u], ibuf)
        pltpu.async_copy(x_hbm.at[ibuf.at[0]], gbuf, gsem0)
        for k in range(FUSE):
            gb, gs = bufs[k % 3]
            if k >= 2:
                pb, ps = bufs[(k - 2) % 3]
                pltpu.make_async_copy(
                    pb, agg.at[ibuf.at[3 * (k - 2) + 1]], ps).wait()
            if k + 1 < FUSE:
                nb, ns = bufs[(k + 1) % 3]
                pltpu.async_copy(x_hbm.at[ibuf.at[3 * (k + 1)]], nb, ns)
            pltpu.make_async_copy(x_hbm.at[ibuf.at[3 * k]], gb, gs).wait()
            scale(gb, 3 * k + 2)
            pltpu.async_copy(gb, agg.at[ibuf.at[3 * k + 1]], gs, add=True)
        for k in (FUSE - 2, FUSE - 1):
            pb, ps = bufs[k % 3]
            pltpu.make_async_copy(pb, agg.at[ibuf.at[3 * k + 1]], ps).wait()
        return carry

    lax.fori_loop(0, NCHUNKS // FUSE, _super, 0)
    plsc.subcore_barrier()

    for i in range((NROW_CHUNKS + NS - 1) // NS):
        cid = s + i * NS

        @pl.when(cid < NROW_CHUNKS)
        def _copy_chunk():
            r0 = pl.multiple_of(cid * CHUNK, CHUNK)
            pltpu.sync_copy(agg.at[pl.ds(r0, CHUNK)],
                            out_hbm.at[c, pl.ds(r0, CHUNK)])


_sc_segment_sum = functools.partial(
    pl.kernel,
    out_type=jax.ShapeDtypeStruct((NC, N, D), jnp.float32),
    mesh=plsc.VectorSubcoreMesh(core_axis_name="c", subcore_axis_name="s"),
    compiler_params=pltpu.CompilerParams(needs_layout_passes=False),
    scratch_types=[
        pltpu.VMEM((3 * FUSE, CHUNK), jnp.int32),
        pltpu.VMEM((CHUNK, D), jnp.float32),
        pltpu.VMEM((CHUNK, D), jnp.float32),
        pltpu.VMEM((CHUNK, D), jnp.float32),
        pltpu.VMEM_SHARED((N, D), jnp.float32),
        pltpu.SemaphoreType.DMA,
        pltpu.SemaphoreType.DMA,
        pltpu.SemaphoreType.DMA,
    ],
)(_sc_body)


BLK = 1000


def _tc_linear_body(p0_ref, p1_ref, wt_ref, b_ref, o_ref):
    acc = p0_ref[...] + p1_ref[...]
    o_ref[...] = (
        jnp.dot(acc, wt_ref[...], preferred_element_type=jnp.float32)
        + b_ref[...]
    )


def _tc_linear(p0, p1, wt, b2):
    return pl.pallas_call(
        _tc_linear_body,
        grid=(N // BLK,),
        in_specs=[
            pl.BlockSpec((BLK, D), lambda i: (i, 0)),
            pl.BlockSpec((BLK, D), lambda i: (i, 0)),
            pl.BlockSpec((D, D), lambda i: (0, 0)),
            pl.BlockSpec((1, D), lambda i: (0, 0)),
        ],
        out_specs=pl.BlockSpec((BLK, D), lambda i: (i, 0)),
        out_shape=jax.ShapeDtypeStruct((N, D), jnp.float32),
    )(p0, p1, wt, b2)


def kernel(X, L_rows, L_cols, L_vals, W, b):
    nsup = NCHUNKS // FUSE
    cols4 = L_cols.reshape(NW, nsup, FUSE, 1, CHUNK)
    rows4 = L_rows.reshape(NW, nsup, FUSE, 1, CHUNK)
    vals4 = jax.lax.bitcast_convert_type(
        L_vals.reshape(NW, nsup, FUSE, 1, CHUNK), jnp.int32)
    idx = jnp.concatenate([cols4, rows4, vals4], axis=3).reshape(
        NW, nsup, 3 * FUSE, CHUNK)
    partials = _sc_segment_sum(X, idx)
    return _tc_linear(partials[0], partials[1], W.T, b.reshape(1, D))
